# baseline (speedup 1.0000x reference)
import functools

import jax
import jax.numpy as jnp
import numpy as np
from jax.experimental import pallas as pl
from jax.experimental.pallas import tpu as pltpu

OUT_SZ = 5
G = OUT_SZ ** 3
R = 16
RB = 8


def _base_grid_t():
    b = np.arange(0, 1, 1.0 / OUT_SZ) - (OUT_SZ - 1) / (2.0 * OUT_SZ)
    gx = np.tile(b.reshape(-1, 1, 1), (1, OUT_SZ, OUT_SZ))
    gy = np.tile(b.reshape(1, -1, 1), (OUT_SZ, 1, OUT_SZ))
    gz = np.tile(b.reshape(1, 1, -1), (OUT_SZ, OUT_SZ, 1))
    g = np.stack([gx, gy, gz], axis=-1).reshape(-1, 3)
    return jnp.asarray(g.T, dtype=jnp.float32)


def _fp_kernel(bg_ref, rois_ref, pts_ref, feats_ref, w_ref,
               h_ref, sum_ref, sq_ref):
    feats = feats_ref[0]
    bg = bg_ref[...]
    Wm = w_ref[...]
    pts = pts_ref[0]
    S = pts.shape[0]
    normp = jnp.sum(pts * pts, axis=1, keepdims=True)
    acc_s = jnp.zeros((Wm.shape[0], 1), jnp.float32)
    acc_q = jnp.zeros((Wm.shape[0], 1), jnp.float32)
    for r in range(R):
        row = rois_ref[0, r]
        ctr_z = row[0:1, 2:3] + 0.5 * row[0:1, 5:6]
        qx = bg[0:1, :] * row[0:1, 3:4] + row[0:1, 0:1]
        qy = bg[1:2, :] * row[0:1, 4:5] + row[0:1, 1:2]
        qz = bg[2:3, :] * row[0:1, 5:6] + ctr_z
        q = jnp.concatenate([qx, qy, qz], axis=0)
        cross = jnp.dot(pts, q, preferred_element_type=jnp.float32)
        normq = jnp.sum(q * q, axis=0, keepdims=True)
        d2 = normq - 2.0 * cross + normp
        CH = 128
        m1 = d2[0:CH]
        m2 = jnp.full((CH, G), jnp.inf, jnp.float32)
        m3 = m2
        for j in range(1, S // CH):
            v = d2[j * CH:(j + 1) * CH]
            t1 = jnp.maximum(m1, v)
            m1 = jnp.minimum(m1, v)
            t2 = jnp.maximum(m2, t1)
            m2 = jnp.minimum(m2, t1)
            m3 = jnp.minimum(m3, t2)
        cand = jnp.concatenate([m1, m2, m3], axis=0)
        mms, recips = [], []
        for k in range(3):
            mm = jnp.min(cand, axis=0, keepdims=True)
            mms.append(mm)
            recips.append(1.0 / (jnp.sqrt(jnp.maximum(mm, 0.0)) + 1e-8))
            if k < 2:
                cand = jnp.where(cand == mm, jnp.inf, cand)
        norm = recips[0] + recips[1] + recips[2]
        a = jnp.where(d2 == mms[0], recips[0] / norm,
                      jnp.where(d2 == mms[1], recips[1] / norm,
                                jnp.where(d2 == mms[2], recips[2] / norm, 0.0)))
        interp = jnp.dot(feats, a, preferred_element_type=jnp.float32)
        h = jnp.dot(Wm, interp, preferred_element_type=jnp.float32)
        h_ref[0, r] = h
        acc_s = acc_s + jnp.sum(h, axis=1, keepdims=True)
        acc_q = acc_q + jnp.sum(h * h, axis=1, keepdims=True)
    sum_ref[0, 0] = acc_s
    sq_ref[0, 0] = acc_q


def _bn_kernel(h_ref, sum_ref, sq_ref, g_ref, b_ref, o_ref, *, count):
    s = sum_ref[...]
    qq = sq_ref[...]
    mean = s / count
    var = qq / count - mean * mean
    scale = g_ref[...] * jax.lax.rsqrt(var + 1e-5)
    shift = b_ref[...] - mean * scale
    for r in range(RB):
        o_ref[r] = jnp.maximum(h_ref[0, r] * scale + shift, 0.0)


def kernel(pts, pts_feature, rois, W, gamma, beta):
    B, S, _ = pts.shape
    C = pts_feature.shape[1]
    N = rois.shape[1]
    NT = N // R
    rois4 = rois.reshape(B, N, 1, 7)
    bg = _base_grid_t()

    h4, psum, psq = pl.pallas_call(
        _fp_kernel,
        grid=(B, NT),
        in_specs=[
            pl.BlockSpec((3, G), lambda b, n: (0, 0)),
            pl.BlockSpec((1, R, 1, 7), lambda b, n: (b, n, 0, 0)),
            pl.BlockSpec((1, S, 3), lambda b, n: (b, 0, 0)),
            pl.BlockSpec((1, C, S), lambda b, n: (b, 0, 0)),
            pl.BlockSpec((C, C), lambda b, n: (0, 0)),
        ],
        out_specs=[
            pl.BlockSpec((1, R, C, G), lambda b, n: (b, n, 0, 0)),
            pl.BlockSpec((1, 1, C, 1), lambda b, n: (b, n, 0, 0)),
            pl.BlockSpec((1, 1, C, 1), lambda b, n: (b, n, 0, 0)),
        ],
        out_shape=[
            jax.ShapeDtypeStruct((B, N, C, G), jnp.float32),
            jax.ShapeDtypeStruct((B, NT, C, 1), jnp.float32),
            jax.ShapeDtypeStruct((B, NT, C, 1), jnp.float32),
        ],
    )(bg, rois4, pts, pts_feature, W)

    ssum = jnp.sum(psum, axis=(0, 1))
    ssq = jnp.sum(psq, axis=(0, 1))

    NB = N // RB
    out = pl.pallas_call(
        functools.partial(_bn_kernel, count=float(B * N * G)),
        grid=(B, NB),
        in_specs=[
            pl.BlockSpec((1, RB, C, G), lambda b, n: (b, n, 0, 0)),
            pl.BlockSpec((C, 1), lambda b, n: (0, 0)),
            pl.BlockSpec((C, 1), lambda b, n: (0, 0)),
            pl.BlockSpec((C, 1), lambda b, n: (0, 0)),
            pl.BlockSpec((C, 1), lambda b, n: (0, 0)),
        ],
        out_specs=pl.BlockSpec((RB, C, G),
                               lambda b, n, NB=NB: (b * NB + n, 0, 0)),
        out_shape=jax.ShapeDtypeStruct((B * N, C, G), jnp.float32),
    )(h4, ssum, ssq, gamma.reshape(C, 1), beta.reshape(C, 1))
    return out
